# Initial kernel scaffold; baseline (speedup 1.0000x reference)
import jax
import jax.numpy as jnp
from jax.experimental import pallas as pl


def kernel(x, edge_index, W_iou, U_iou, b_iou, W_f, U_f, b_f, clf_W1, clf_b1, clf_W2, clf_b2):
    raise NotImplementedError("write your pallas kernel here")


# speedup vs baseline: 2.0863x; 2.0863x over previous
import functools

import jax
import jax.numpy as jnp
from jax import lax
from jax.experimental import pallas as pl
from jax.experimental.pallas import tpu as pltpu
from jax.experimental.pallas import tpu_sc as plsc

F32 = jnp.float32
NC, NS, LANES = 2, 16, 16
K = 80
ZC = 80


def _p1_body(x_ref, Wf_ref, Uf_ref, bf_ref, Wiou_ref, biou_ref,
             b_ref, ap_ref, iou0_ref):
    xv = x_ref[...]
    b_ref[...] = jnp.dot(xv, Uf_ref[...], preferred_element_type=F32)
    ap_ref[...] = jnp.dot(xv, Wf_ref[...], preferred_element_type=F32) + bf_ref[...]
    iou0_ref[...] = jnp.dot(xv, Wiou_ref[...], preferred_element_type=F32) + biou_ref[...]


def _p3_body(m_ref, c_ref, iou0_ref, Uiou_ref, W1a_ref, W1b_ref, b1_ref,
             p_ref, q_ref):
    H = m_ref.shape[1]
    iou = iou0_ref[...] + jnp.dot(m_ref[...], Uiou_ref[...], preferred_element_type=F32)
    i = jax.nn.sigmoid(iou[:, :H])
    o = jax.nn.sigmoid(iou[:, H:2 * H])
    u = jnp.tanh(iou[:, 2 * H:])
    c = i * u + c_ref[...]
    h = o * jnp.tanh(c)
    p_ref[...] = jnp.dot(h, W1a_ref[...], preferred_element_type=F32)
    q_ref[...] = jnp.dot(h, W1b_ref[...], preferred_element_type=F32) + b1_ref[...]


def _p5_body(g_ref, W2_ref, b2_ref, out_ref):
    out_ref[...] = jnp.dot(jnp.tanh(g_ref[...]), W2_ref[...],
                           preferred_element_type=F32) + b2_ref[...]


def _sc1_body(x_hbm, b_hbm, ap_hbm, src_hbm, dst_hbm, m_hbm, c_hbm,
              idx_s, idx_d, x_v, b_v, a_v, o_v, acc_sh):
    cid = lax.axis_index("c")
    sid = lax.axis_index("s")
    E = src_hbm.shape[0]
    N, H = m_hbm.shape
    ept = E // NS
    nb = ept // K
    nch = N // ZC
    nq = H // LANES

    def _zb(i, _):
        r = i // nq
        q = i % nq
        o_v[r, pl.ds(q * LANES, LANES)] = jnp.zeros((LANES,), F32)
        return 0
    lax.fori_loop(0, ZC * nq, _zb, 0)

    def _zr(k, _):
        ch = k * NS + sid
        @pl.when(ch < nch)
        def _():
            pltpu.sync_copy(o_v, acc_sh.at[pl.ds(ch * ZC, ZC)])
        return 0
    lax.fori_loop(0, (nch + NS - 1) // NS, _zr, 0)
    plsc.subcore_barrier()

    def _blk(b, _):
        base = sid * ept + b * K
        pltpu.sync_copy(src_hbm.at[pl.ds(base, K)], idx_s.at[0])
        pltpu.sync_copy(dst_hbm.at[pl.ds(base, K)], idx_d.at[0])

        @pl.when(cid == 0)
        def _():
            pltpu.sync_copy(x_hbm.at[idx_s.at[0]], o_v)
            pltpu.sync_copy(o_v, acc_sh.at[idx_d.at[0]], add=True)

        @pl.when(cid == 1)
        def _():
            pltpu.sync_copy(x_hbm.at[idx_s.at[0]], x_v)
            pltpu.sync_copy(b_hbm.at[idx_s.at[0]], b_v)
            pltpu.sync_copy(ap_hbm.at[idx_d.at[0]], a_v)

            def _ed(j, _):
                for q in range(nq):
                    sl = pl.ds(q * LANES, LANES)
                    z = a_v[j, sl] + b_v[j, sl]
                    f = 1.0 / (1.0 + jnp.exp(-z))
                    o_v[j, sl] = f * x_v[j, sl]
                return 0
            lax.fori_loop(0, K, _ed, 0)
            pltpu.sync_copy(o_v, acc_sh.at[idx_d.at[0]], add=True)
        return 0
    lax.fori_loop(0, nb, _blk, 0)
    plsc.subcore_barrier()

    def _ex(k, _):
        ch = k * NS + sid
        @pl.when(ch < nch)
        def _():
            rows = pl.ds(ch * ZC, ZC)
            pltpu.sync_copy(acc_sh.at[rows], o_v)

            @pl.when(cid == 0)
            def _():
                pltpu.sync_copy(o_v, m_hbm.at[rows])

            @pl.when(cid == 1)
            def _():
                pltpu.sync_copy(o_v, c_hbm.at[rows])
        return 0
    lax.fori_loop(0, (nch + NS - 1) // NS, _ex, 0)


def _sc2_body(p_hbm, q_hbm, src_hbm, dst_hbm, g_hbm, idx_s, idx_d, p_v, q_v):
    cid = lax.axis_index("c")
    sid = lax.axis_index("s")
    wid = sid * NC + cid
    E = src_hbm.shape[0]
    H = p_hbm.shape[1]
    ept = E // (NC * NS)
    nb = ept // K
    nq = H // LANES

    def _blk(b, _):
        base = wid * ept + b * K
        pltpu.sync_copy(src_hbm.at[pl.ds(base, K)], idx_s.at[0])
        pltpu.sync_copy(dst_hbm.at[pl.ds(base, K)], idx_d.at[0])
        pltpu.sync_copy(p_hbm.at[idx_s.at[0]], p_v)
        pltpu.sync_copy(q_hbm.at[idx_d.at[0]], q_v)

        def _ed(j, _):
            for q in range(nq):
                sl = pl.ds(q * LANES, LANES)
                p_v[j, sl] = p_v[j, sl] + q_v[j, sl]
            return 0
        lax.fori_loop(0, K, _ed, 0)
        pltpu.sync_copy(p_v, g_hbm.at[pl.ds(base, K)])
        return 0
    lax.fori_loop(0, nb, _blk, 0)


def kernel(x, edge_index, W_iou, U_iou, b_iou, W_f, U_f, b_f,
           clf_W1, clf_b1, clf_W2, clf_b2):
    N, D = x.shape
    H = U_f.shape[0]
    E = edge_index.shape[1]
    src = edge_index[0]
    dst = edge_index[1]

    bf_r = b_f.reshape(1, H)
    biou_r = b_iou.reshape(1, 3 * H)
    b1_r = clf_b1.reshape(1, -1)
    W1a = clf_W1[:H]
    W1b = clf_W1[H:]
    OUTP = 8
    W2p = jnp.pad(clf_W2, ((0, 0), (0, OUTP - clf_W2.shape[1])))
    b2p = jnp.pad(clf_b2, (0, OUTP - clf_b2.shape[0])).reshape(1, OUTP)

    R1 = 1000
    full = lambda s: pl.BlockSpec(s, lambda i: (0, 0))
    B, Ap, iou0 = pl.pallas_call(
        _p1_body,
        grid=(N // R1,),
        in_specs=[
            pl.BlockSpec((R1, D), lambda i: (i, 0)),
            full((D, H)), full((H, H)), full((1, H)),
            full((D, 3 * H)), full((1, 3 * H)),
        ],
        out_specs=[
            pl.BlockSpec((R1, H), lambda i: (i, 0)),
            pl.BlockSpec((R1, H), lambda i: (i, 0)),
            pl.BlockSpec((R1, 3 * H), lambda i: (i, 0)),
        ],
        out_shape=[
            jax.ShapeDtypeStruct((N, H), F32),
            jax.ShapeDtypeStruct((N, H), F32),
            jax.ShapeDtypeStruct((N, 3 * H), F32),
        ],
    )(x, W_f, U_f, bf_r, W_iou, biou_r)

    mesh = plsc.VectorSubcoreMesh(core_axis_name="c", subcore_axis_name="s",
                                  num_cores=NC, num_subcores=NS)
    m, c_sum = pl.kernel(
        _sc1_body,
        out_type=(jax.ShapeDtypeStruct((N, H), F32),
                  jax.ShapeDtypeStruct((N, H), F32)),
        mesh=mesh,
        scratch_types=[
            pltpu.VMEM((1, K), jnp.int32),
            pltpu.VMEM((1, K), jnp.int32),
            pltpu.VMEM((K, H), F32),
            pltpu.VMEM((K, H), F32),
            pltpu.VMEM((K, H), F32),
            pltpu.VMEM((K, H), F32),
            pltpu.VMEM_SHARED((N, H), F32),
        ],
    )(x, B, Ap, src, dst)

    P, Q = pl.pallas_call(
        _p3_body,
        grid=(N // R1,),
        in_specs=[
            pl.BlockSpec((R1, H), lambda i: (i, 0)),
            pl.BlockSpec((R1, H), lambda i: (i, 0)),
            pl.BlockSpec((R1, 3 * H), lambda i: (i, 0)),
            full((H, 3 * H)), full((H, H)), full((H, H)), full((1, H)),
        ],
        out_specs=[
            pl.BlockSpec((R1, H), lambda i: (i, 0)),
            pl.BlockSpec((R1, H), lambda i: (i, 0)),
        ],
        out_shape=[
            jax.ShapeDtypeStruct((N, H), F32),
            jax.ShapeDtypeStruct((N, H), F32),
        ],
    )(m, c_sum, iou0, U_iou, W1a, W1b, b1_r)

    G = pl.kernel(
        _sc2_body,
        out_type=jax.ShapeDtypeStruct((E, H), F32),
        mesh=mesh,
        scratch_types=[
            pltpu.VMEM((1, K), jnp.int32),
            pltpu.VMEM((1, K), jnp.int32),
            pltpu.VMEM((K, H), F32),
            pltpu.VMEM((K, H), F32),
        ],
    )(P, Q, src, dst)

    RG = 4000
    out8 = pl.pallas_call(
        _p5_body,
        grid=(E // RG,),
        in_specs=[
            pl.BlockSpec((RG, H), lambda i: (i, 0)),
            full((H, OUTP)), full((1, OUTP)),
        ],
        out_specs=pl.BlockSpec((RG, OUTP), lambda i: (i, 0)),
        out_shape=jax.ShapeDtypeStruct((E, OUTP), F32),
    )(G, W2p, b2p)

    return out8[:, :3]


# speedup vs baseline: 3.6969x; 1.7720x over previous
import jax
import jax.numpy as jnp
from jax import lax
from jax.experimental import pallas as pl
from jax.experimental.pallas import tpu as pltpu
from jax.experimental.pallas import tpu_sc as plsc

F32 = jnp.float32
NC, NS, LANES = 2, 16, 16
K = 40


def _p1_body(x_ref, Wf_ref, Uf_ref, bf_ref, Wiou_ref, biou_ref,
             b_ref, ap_ref, iou0_ref):
    xv = x_ref[...]
    b_ref[...] = jnp.dot(xv, Uf_ref[...], preferred_element_type=F32)
    ap_ref[...] = jnp.dot(xv, Wf_ref[...], preferred_element_type=F32) + bf_ref[...]
    iou0_ref[...] = jnp.dot(xv, Wiou_ref[...], preferred_element_type=F32) + biou_ref[...]


def _p3_body(m_ref, c_ref, iou0_ref, Uiou_ref, W1a_ref, W1b_ref, b1_ref,
             p_ref, q_ref):
    H = m_ref.shape[1]
    iou = iou0_ref[...] + jnp.dot(m_ref[...], Uiou_ref[...], preferred_element_type=F32)
    i = jax.nn.sigmoid(iou[:, :H])
    o = jax.nn.sigmoid(iou[:, H:2 * H])
    u = jnp.tanh(iou[:, 2 * H:])
    c = i * u + c_ref[...]
    h = o * jnp.tanh(c)
    p_ref[...] = jnp.dot(h, W1a_ref[...], preferred_element_type=F32)
    q_ref[...] = jnp.dot(h, W1b_ref[...], preferred_element_type=F32) + b1_ref[...]


def _p5_body(g_ref, W2_ref, b2_ref, out_ref):
    out_ref[...] = jnp.dot(jnp.tanh(g_ref[...]), W2_ref[...],
                           preferred_element_type=F32) + b2_ref[...]


def _sc1_body(x_hbm, b_hbm, ap_hbm, src_hbm, dst_hbm, m_hbm, c_hbm,
              idx_s0, idx_d0, idx_s1, idx_d1, idx_s2, idx_d2, idx_s3, idx_d3,
              x_v0, x_v1, b_v0, b_v1, a_v0, a_v1, acc_sh,
              sg0, sg1, ss0, ss1, si0, si1, si2, si3):
    cid = lax.axis_index("c")
    sid = lax.axis_index("s")
    E = src_hbm.shape[0]
    N, H = m_hbm.shape
    ept = E // NS
    nb = ept // K
    nch = N // K
    nq = H // LANES
    ebase = sid * ept
    idx_s = (idx_s0, idx_s1, idx_s2, idx_s3)
    idx_d = (idx_d0, idx_d1, idx_d2, idx_d3)
    x_v = (x_v0, x_v1)
    b_v = (b_v0, b_v1)
    a_v = (a_v0, a_v1)
    sg = (sg0, sg1)
    ss = (ss0, ss1)
    si = (si0, si1, si2, si3)

    def _zb(i, _):
        r = i // nq
        q = i % nq
        x_v0[r, pl.ds(q * LANES, LANES)] = jnp.zeros((LANES,), F32)
        return 0
    lax.fori_loop(0, K * nq, _zb, 0)

    def _zr(k, _):
        ch = k * NS + sid
        @pl.when(ch < nch)
        def _():
            pltpu.sync_copy(x_v0, acc_sh.at[pl.ds(ch * K, K)])
        return 0
    lax.fori_loop(0, (nch + NS - 1) // NS, _zr, 0)
    plsc.subcore_barrier()

    def issue_idx(b, q):
        off = ebase + b * K
        pltpu.async_copy(src_hbm.at[pl.ds(off, K)], idx_s[q].at[0], si[q])
        pltpu.async_copy(dst_hbm.at[pl.ds(off, K)], idx_d[q].at[0], si[q])

    def wait_idx(q):
        pltpu.make_async_copy(src_hbm.at[pl.ds(0, K)], idx_s[q].at[0], si[q]).wait()
        pltpu.make_async_copy(dst_hbm.at[pl.ds(0, K)], idx_d[q].at[0], si[q]).wait()

    def issue_gathers(dp, q):
        pltpu.async_copy(x_hbm.at[idx_s[q].at[0]], x_v[dp], sg[dp])
        @pl.when(cid == 1)
        def _():
            pltpu.async_copy(b_hbm.at[idx_s[q].at[0]], b_v[dp], sg[dp])
            pltpu.async_copy(ap_hbm.at[idx_d[q].at[0]], a_v[dp], sg[dp])

    def wait_gathers(dp, q):
        pltpu.make_async_copy(x_hbm.at[idx_s[q].at[0]], x_v[dp], sg[dp]).wait()
        @pl.when(cid == 1)
        def _():
            pltpu.make_async_copy(b_hbm.at[idx_s[q].at[0]], b_v[dp], sg[dp]).wait()
            pltpu.make_async_copy(ap_hbm.at[idx_d[q].at[0]], a_v[dp], sg[dp]).wait()

    def issue_scatter(dp, q):
        pltpu.async_copy(x_v[dp], acc_sh.at[idx_d[q].at[0]], ss[dp], add=True)

    def wait_scatter(dp):
        pltpu.make_async_copy(x_v[dp], acc_sh.at[idx_d[0].at[0]], ss[dp]).wait()

    def compute(dp):
        @pl.when(cid == 1)
        def _():
            @plsc.parallel_loop(0, K)
            def _ed(j):
                for q in range(nq):
                    sl = pl.ds(q * LANES, LANES)
                    z = a_v[dp][j, sl] + b_v[dp][j, sl]
                    f = 1.0 / (1.0 + jnp.exp(-z))
                    x_v[dp][j, sl] = f * x_v[dp][j, sl]

    issue_idx(0, 0)
    wait_idx(0)
    issue_gathers(0, 0)
    issue_idx(1, 1)

    def _g(g, _):
        for p in (0, 1, 2, 3):
            b = 4 * g + p
            dp = p % 2
            wait_gathers(dp, p)

            @pl.when(b + 2 < nb)
            def _():
                issue_idx(b + 2, (p + 2) % 4)

            @pl.when(b + 1 < nb)
            def _():
                wait_idx((p + 1) % 4)
                @pl.when(b >= 1)
                def _():
                    wait_scatter(1 - dp)
                issue_gathers(1 - dp, (p + 1) % 4)

            compute(dp)
            issue_scatter(dp, p)
        return 0
    lax.fori_loop(0, nb // 4, _g, 0)
    wait_scatter(0)
    wait_scatter(1)
    plsc.subcore_barrier()

    def _ex(k, _):
        ch = k * NS + sid
        @pl.when(ch < nch)
        def _():
            rows = pl.ds(ch * K, K)
            pltpu.sync_copy(acc_sh.at[rows], x_v0)

            @pl.when(cid == 0)
            def _():
                pltpu.sync_copy(x_v0, m_hbm.at[rows])

            @pl.when(cid == 1)
            def _():
                pltpu.sync_copy(x_v0, c_hbm.at[rows])
        return 0
    lax.fori_loop(0, (nch + NS - 1) // NS, _ex, 0)


def _sc2_body(p_hbm, q_hbm, src_hbm, dst_hbm, g_hbm,
              idx_s0, idx_d0, idx_s1, idx_d1,
              p_v0, p_v1, q_v0, q_v1,
              sg0, sg1, sw0, sw1, si0, si1):
    cid = lax.axis_index("c")
    sid = lax.axis_index("s")
    wid = sid * NC + cid
    E = src_hbm.shape[0]
    H = p_hbm.shape[1]
    ept = E // (NC * NS)
    nb = ept // K
    nq = H // LANES
    ebase = wid * ept
    idx_s = (idx_s0, idx_s1)
    idx_d = (idx_d0, idx_d1)
    p_v = (p_v0, p_v1)
    q_v = (q_v0, q_v1)
    sg = (sg0, sg1)
    sw = (sw0, sw1)
    si = (si0, si1)

    def issue_idx(b, p):
        off = ebase + b * K
        pltpu.async_copy(src_hbm.at[pl.ds(off, K)], idx_s[p].at[0], si[p])
        pltpu.async_copy(dst_hbm.at[pl.ds(off, K)], idx_d[p].at[0], si[p])

    def wait_idx(p):
        pltpu.make_async_copy(src_hbm.at[pl.ds(0, K)], idx_s[p].at[0], si[p]).wait()
        pltpu.make_async_copy(dst_hbm.at[pl.ds(0, K)], idx_d[p].at[0], si[p]).wait()

    def issue_gathers(p):
        pltpu.async_copy(p_hbm.at[idx_s[p].at[0]], p_v[p], sg[p])
        pltpu.async_copy(q_hbm.at[idx_d[p].at[0]], q_v[p], sg[p])

    def wait_gathers(p):
        pltpu.make_async_copy(p_hbm.at[idx_s[p].at[0]], p_v[p], sg[p]).wait()
        pltpu.make_async_copy(q_hbm.at[idx_d[p].at[0]], q_v[p], sg[p]).wait()

    def issue_write(b, p):
        pltpu.async_copy(p_v[p], g_hbm.at[pl.ds(ebase + b * K, K)], sw[p])

    def wait_write(p):
        pltpu.make_async_copy(p_v[p], g_hbm.at[pl.ds(ebase, K)], sw[p]).wait()

    def compute(p):
        @plsc.parallel_loop(0, K)
        def _ed(j):
            for q in range(nq):
                sl = pl.ds(q * LANES, LANES)
                p_v[p][j, sl] = p_v[p][j, sl] + q_v[p][j, sl]

    issue_idx(0, 0)
    wait_idx(0)
    issue_gathers(0)
    issue_idx(1, 1)

    def _g(g, _):
        for p in (0, 1):
            b = 2 * g + p
            wait_gathers(p)

            @pl.when(b + 2 < nb)
            def _():
                issue_idx(b + 2, p)

            @pl.when(b + 1 < nb)
            def _():
                wait_idx(1 - p)
                @pl.when(b >= 1)
                def _():
                    wait_write(1 - p)
                issue_gathers(1 - p)

            compute(p)
            issue_write(b, p)
        return 0
    lax.fori_loop(0, nb // 2, _g, 0)
    wait_write(0)
    wait_write(1)


def kernel(x, edge_index, W_iou, U_iou, b_iou, W_f, U_f, b_f,
           clf_W1, clf_b1, clf_W2, clf_b2):
    N, D = x.shape
    H = U_f.shape[0]
    E = edge_index.shape[1]
    src = edge_index[0]
    dst = edge_index[1]

    bf_r = b_f.reshape(1, H)
    biou_r = b_iou.reshape(1, 3 * H)
    b1_r = clf_b1.reshape(1, -1)
    W1a = clf_W1[:H]
    W1b = clf_W1[H:]
    OUTP = 8
    W2p = jnp.pad(clf_W2, ((0, 0), (0, OUTP - clf_W2.shape[1])))
    b2p = jnp.pad(clf_b2, (0, OUTP - clf_b2.shape[0])).reshape(1, OUTP)

    R1 = 1000
    full = lambda s: pl.BlockSpec(s, lambda i: (0, 0))
    B, Ap, iou0 = pl.pallas_call(
        _p1_body,
        grid=(N // R1,),
        in_specs=[
            pl.BlockSpec((R1, D), lambda i: (i, 0)),
            full((D, H)), full((H, H)), full((1, H)),
            full((D, 3 * H)), full((1, 3 * H)),
        ],
        out_specs=[
            pl.BlockSpec((R1, H), lambda i: (i, 0)),
            pl.BlockSpec((R1, H), lambda i: (i, 0)),
            pl.BlockSpec((R1, 3 * H), lambda i: (i, 0)),
        ],
        out_shape=[
            jax.ShapeDtypeStruct((N, H), F32),
            jax.ShapeDtypeStruct((N, H), F32),
            jax.ShapeDtypeStruct((N, 3 * H), F32),
        ],
    )(x, W_f, U_f, bf_r, W_iou, biou_r)

    mesh = plsc.VectorSubcoreMesh(core_axis_name="c", subcore_axis_name="s",
                                  num_cores=NC, num_subcores=NS)
    m, c_sum = pl.kernel(
        _sc1_body,
        out_type=(jax.ShapeDtypeStruct((N, H), F32),
                  jax.ShapeDtypeStruct((N, H), F32)),
        mesh=mesh,
        scratch_types=(
            [pltpu.VMEM((1, K), jnp.int32)] * 8
            + [pltpu.VMEM((K, H), F32)] * 6
            + [pltpu.VMEM_SHARED((N, H), F32)]
            + [pltpu.SemaphoreType.DMA] * 8
        ),
    )(x, B, Ap, src, dst)

    P, Q = pl.pallas_call(
        _p3_body,
        grid=(N // R1,),
        in_specs=[
            pl.BlockSpec((R1, H), lambda i: (i, 0)),
            pl.BlockSpec((R1, H), lambda i: (i, 0)),
            pl.BlockSpec((R1, 3 * H), lambda i: (i, 0)),
            full((H, 3 * H)), full((H, H)), full((H, H)), full((1, H)),
        ],
        out_specs=[
            pl.BlockSpec((R1, H), lambda i: (i, 0)),
            pl.BlockSpec((R1, H), lambda i: (i, 0)),
        ],
        out_shape=[
            jax.ShapeDtypeStruct((N, H), F32),
            jax.ShapeDtypeStruct((N, H), F32),
        ],
    )(m, c_sum, iou0, U_iou, W1a, W1b, b1_r)

    G = pl.kernel(
        _sc2_body,
        out_type=jax.ShapeDtypeStruct((E, H), F32),
        mesh=mesh,
        scratch_types=[
            pltpu.VMEM((1, K), jnp.int32),
            pltpu.VMEM((1, K), jnp.int32),
            pltpu.VMEM((1, K), jnp.int32),
            pltpu.VMEM((1, K), jnp.int32),
            pltpu.VMEM((K, H), F32),
            pltpu.VMEM((K, H), F32),
            pltpu.VMEM((K, H), F32),
            pltpu.VMEM((K, H), F32),
            pltpu.SemaphoreType.DMA,
            pltpu.SemaphoreType.DMA,
            pltpu.SemaphoreType.DMA,
            pltpu.SemaphoreType.DMA,
            pltpu.SemaphoreType.DMA,
            pltpu.SemaphoreType.DMA,
        ],
    )(P, Q, src, dst)

    RG = 4000
    out8 = pl.pallas_call(
        _p5_body,
        grid=(E // RG,),
        in_specs=[
            pl.BlockSpec((RG, H), lambda i: (i, 0)),
            full((H, OUTP)), full((1, OUTP)),
        ],
        out_specs=pl.BlockSpec((RG, OUTP), lambda i: (i, 0)),
        out_shape=jax.ShapeDtypeStruct((E, OUTP), F32),
    )(G, W2p, b2p)

    return out8[:, :3]


# speedup vs baseline: 4.0203x; 1.0875x over previous
import jax
import jax.numpy as jnp
from jax import lax
from jax.experimental import pallas as pl
from jax.experimental.pallas import tpu as pltpu
from jax.experimental.pallas import tpu_sc as plsc

F32 = jnp.float32
NC, NS, LANES = 2, 16, 16
K = 40


def _p1_body(x_ref, Wf_ref, Uf_ref, bf_ref, Wiou_ref, biou_ref,
             xbl_ref, xbr_ref, ap_ref, iou0_ref):
    xv = x_ref[...]
    H2 = xv.shape[1] // 2
    bv = jnp.dot(xv, Uf_ref[...], preferred_element_type=F32)
    xbl_ref[:, :H2] = xv[:, :H2]
    xbl_ref[:, H2:] = bv[:, :H2]
    xbr_ref[:, :H2] = xv[:, H2:]
    xbr_ref[:, H2:] = bv[:, H2:]
    ap_ref[...] = jnp.dot(xv, Wf_ref[...], preferred_element_type=F32) + bf_ref[...]
    iou0_ref[...] = jnp.dot(xv, Wiou_ref[...], preferred_element_type=F32) + biou_ref[...]


def _p3_body(mcl_ref, mcr_ref, iou0_ref, UiouL_ref, UiouR_ref,
             W1a_ref, W1b_ref, b1_ref, p_ref, q_ref):
    H = mcl_ref.shape[1]
    H2 = H // 2
    mcl = mcl_ref[...]
    mcr = mcr_ref[...]
    c_ref = jnp.concatenate([mcl[:, H2:], mcr[:, H2:]], axis=1)
    iou = (iou0_ref[...]
           + jnp.dot(mcl[:, :H2], UiouL_ref[...], preferred_element_type=F32)
           + jnp.dot(mcr[:, :H2], UiouR_ref[...], preferred_element_type=F32))
    i = jax.nn.sigmoid(iou[:, :H])
    o = jax.nn.sigmoid(iou[:, H:2 * H])
    u = jnp.tanh(iou[:, 2 * H:])
    c = i * u + c_ref
    h = o * jnp.tanh(c)
    p_ref[...] = jnp.dot(h, W1a_ref[...], preferred_element_type=F32)
    q_ref[...] = jnp.dot(h, W1b_ref[...], preferred_element_type=F32) + b1_ref[...]


def _p5_body(g_ref, W2_ref, b2_ref, out_ref):
    out_ref[...] = jnp.dot(jnp.tanh(g_ref[...]), W2_ref[...],
                           preferred_element_type=F32) + b2_ref[...]


def _sc1_body(xbl_hbm, xbr_hbm, ap_hbm, src_hbm, dst_hbm,
              mcl_hbm, mcr_hbm, *scr):
    idx_s = scr[0:4]
    idx_d = scr[4:8]
    xb_v = scr[8:12]
    a_v = scr[12:16]
    acc = scr[16]
    si = scr[17:21]
    sg = scr[21:25]
    ss = scr[25:29]
    cid = lax.axis_index("c")
    sid = lax.axis_index("s")
    E = src_hbm.shape[0]
    N, H = mcl_hbm.shape
    H2 = H // 2
    ept = E // NS
    nb = ept // K
    nch = N // K
    nq = H2 // LANES
    ebase = sid * ept
    acol = cid * H2

    def _zb(i, _):
        r = i // (2 * nq)
        q = i % (2 * nq)
        xb_v[0][r, pl.ds(q * LANES, LANES)] = jnp.zeros((LANES,), F32)
        return 0
    lax.fori_loop(0, K * 2 * nq, _zb, 0)

    def _zr(k, _):
        ch = k * NS + sid
        @pl.when(ch < nch)
        def _():
            pltpu.sync_copy(xb_v[0], acc.at[pl.ds(ch * K, K)])
        return 0
    lax.fori_loop(0, (nch + NS - 1) // NS, _zr, 0)
    plsc.subcore_barrier()

    def issue_idx(b, q):
        off = ebase + b * K
        pltpu.async_copy(src_hbm.at[pl.ds(off, K)], idx_s[q].at[0], si[q])
        pltpu.async_copy(dst_hbm.at[pl.ds(off, K)], idx_d[q].at[0], si[q])

    def wait_idx(q):
        pltpu.make_async_copy(src_hbm.at[pl.ds(0, K)], idx_s[q].at[0], si[q]).wait()
        pltpu.make_async_copy(dst_hbm.at[pl.ds(0, K)], idx_d[q].at[0], si[q]).wait()

    def issue_gathers(t):
        pltpu.async_copy(ap_hbm.at[idx_d[t].at[0]], a_v[t], sg[t])
        @pl.when(cid == 0)
        def _():
            pltpu.async_copy(xbl_hbm.at[idx_s[t].at[0]], xb_v[t], sg[t])
        @pl.when(cid == 1)
        def _():
            pltpu.async_copy(xbr_hbm.at[idx_s[t].at[0]], xb_v[t], sg[t])

    def wait_gathers(t):
        pltpu.make_async_copy(ap_hbm.at[idx_d[t].at[0]], a_v[t], sg[t]).wait()
        pltpu.make_async_copy(xbl_hbm.at[idx_s[t].at[0]], xb_v[t], sg[t]).wait()

    def issue_scatters(t):
        pltpu.async_copy(xb_v[t], acc.at[idx_d[t].at[0]], ss[t], add=True)

    def wait_scatters(t):
        pltpu.make_async_copy(xb_v[t], acc.at[idx_d[t].at[0]], ss[t]).wait()

    def compute(t):
        @plsc.parallel_loop(0, K)
        def _ed(j):
            for q in range(nq):
                slx = pl.ds(q * LANES, LANES)
                slb = pl.ds(H2 + q * LANES, LANES)
                z = a_v[t][j, pl.ds(acol + q * LANES, LANES)] + xb_v[t][j, slb]
                f = 1.0 / (1.0 + jnp.exp(-z))
                xb_v[t][j, slb] = f * xb_v[t][j, slx]

    issue_idx(0, 0)
    wait_idx(0)
    issue_gathers(0)
    issue_idx(1, 1)

    def _g(g, _):
        for p in (0, 1, 2, 3):
            b = 4 * g + p
            wait_gathers(p)

            @pl.when(b >= 2)
            def _():
                wait_scatters((p + 2) % 4)

            @pl.when(b + 2 < nb)
            def _():
                issue_idx(b + 2, (p + 2) % 4)

            @pl.when(b + 1 < nb)
            def _():
                wait_idx((p + 1) % 4)
                issue_gathers((p + 1) % 4)

            compute(p)
            issue_scatters(p)
        return 0
    lax.fori_loop(0, nb // 4, _g, 0)
    wait_scatters(2)
    wait_scatters(3)
    plsc.subcore_barrier()

    def _ex(k, _):
        ch = k * NS + sid
        @pl.when(ch < nch)
        def _():
            rows = pl.ds(ch * K, K)
            pltpu.sync_copy(acc.at[rows], xb_v[0])

            @pl.when(cid == 0)
            def _():
                pltpu.sync_copy(xb_v[0], mcl_hbm.at[rows])

            @pl.when(cid == 1)
            def _():
                pltpu.sync_copy(xb_v[0], mcr_hbm.at[rows])
        return 0
    lax.fori_loop(0, (nch + NS - 1) // NS, _ex, 0)


def _sc2_body(p_hbm, q_hbm, src_hbm, dst_hbm, g_hbm,
              idx_s0, idx_d0, idx_s1, idx_d1,
              p_v0, p_v1, q_v0, q_v1,
              sg0, sg1, sw0, sw1, si0, si1):
    cid = lax.axis_index("c")
    sid = lax.axis_index("s")
    wid = sid * NC + cid
    E = src_hbm.shape[0]
    H = p_hbm.shape[1]
    ept = E // (NC * NS)
    nb = ept // K
    nq = H // LANES
    ebase = wid * ept
    idx_s = (idx_s0, idx_s1)
    idx_d = (idx_d0, idx_d1)
    p_v = (p_v0, p_v1)
    q_v = (q_v0, q_v1)
    sg = (sg0, sg1)
    sw = (sw0, sw1)
    si = (si0, si1)

    def issue_idx(b, p):
        off = ebase + b * K
        pltpu.async_copy(src_hbm.at[pl.ds(off, K)], idx_s[p].at[0], si[p])
        pltpu.async_copy(dst_hbm.at[pl.ds(off, K)], idx_d[p].at[0], si[p])

    def wait_idx(p):
        pltpu.make_async_copy(src_hbm.at[pl.ds(0, K)], idx_s[p].at[0], si[p]).wait()
        pltpu.make_async_copy(dst_hbm.at[pl.ds(0, K)], idx_d[p].at[0], si[p]).wait()

    def issue_gathers(p):
        pltpu.async_copy(p_hbm.at[idx_s[p].at[0]], p_v[p], sg[p])
        pltpu.async_copy(q_hbm.at[idx_d[p].at[0]], q_v[p], sg[p])

    def wait_gathers(p):
        pltpu.make_async_copy(p_hbm.at[idx_s[p].at[0]], p_v[p], sg[p]).wait()
        pltpu.make_async_copy(q_hbm.at[idx_d[p].at[0]], q_v[p], sg[p]).wait()

    def issue_write(b, p):
        pltpu.async_copy(p_v[p], g_hbm.at[pl.ds(ebase + b * K, K)], sw[p])

    def wait_write(p):
        pltpu.make_async_copy(p_v[p], g_hbm.at[pl.ds(ebase, K)], sw[p]).wait()

    def compute(p):
        @plsc.parallel_loop(0, K)
        def _ed(j):
            for q in range(nq):
                sl = pl.ds(q * LANES, LANES)
                p_v[p][j, sl] = p_v[p][j, sl] + q_v[p][j, sl]

    issue_idx(0, 0)
    wait_idx(0)
    issue_gathers(0)
    issue_idx(1, 1)

    def _g(g, _):
        for p in (0, 1):
            b = 2 * g + p
            wait_gathers(p)

            @pl.when(b + 2 < nb)
            def _():
                issue_idx(b + 2, p)

            @pl.when(b + 1 < nb)
            def _():
                wait_idx(1 - p)
                @pl.when(b >= 1)
                def _():
                    wait_write(1 - p)
                issue_gathers(1 - p)

            compute(p)
            issue_write(b, p)
        return 0
    lax.fori_loop(0, nb // 2, _g, 0)
    wait_write(0)
    wait_write(1)


def kernel(x, edge_index, W_iou, U_iou, b_iou, W_f, U_f, b_f,
           clf_W1, clf_b1, clf_W2, clf_b2):
    N, D = x.shape
    H = U_f.shape[0]
    E = edge_index.shape[1]
    src = edge_index[0]
    dst = edge_index[1]

    bf_r = b_f.reshape(1, H)
    biou_r = b_iou.reshape(1, 3 * H)
    b1_r = clf_b1.reshape(1, -1)
    W1a = clf_W1[:H]
    W1b = clf_W1[H:]
    OUTP = 8
    W2p = jnp.pad(clf_W2, ((0, 0), (0, OUTP - clf_W2.shape[1])))
    b2p = jnp.pad(clf_b2, (0, OUTP - clf_b2.shape[0])).reshape(1, OUTP)

    R1 = 1000
    H2 = H // 2
    full = lambda s: pl.BlockSpec(s, lambda i: (0, 0))
    rowblk = lambda w: pl.BlockSpec((R1, w), lambda i: (i, 0))
    node_t = jax.ShapeDtypeStruct((N, H), F32)
    XBL, XBR, Ap, iou0 = pl.pallas_call(
        _p1_body,
        grid=(N // R1,),
        in_specs=[
            rowblk(D),
            full((D, H)), full((H, H)), full((1, H)),
            full((D, 3 * H)), full((1, 3 * H)),
        ],
        out_specs=[rowblk(H)] * 3 + [rowblk(3 * H)],
        out_shape=[node_t] * 3 + [jax.ShapeDtypeStruct((N, 3 * H), F32)],
    )(x, W_f, U_f, bf_r, W_iou, biou_r)

    mesh = plsc.VectorSubcoreMesh(core_axis_name="c", subcore_axis_name="s",
                                  num_cores=NC, num_subcores=NS)
    mcL, mcR = pl.kernel(
        _sc1_body,
        out_type=(node_t, node_t),
        mesh=mesh,
        scratch_types=(
            [pltpu.VMEM((1, K), jnp.int32)] * 8
            + [pltpu.VMEM((K, H), F32)] * 8
            + [pltpu.VMEM_SHARED((N, H), F32)]
            + [pltpu.SemaphoreType.DMA] * 12
        ),
    )(XBL, XBR, Ap, src, dst)

    P, Q = pl.pallas_call(
        _p3_body,
        grid=(N // R1,),
        in_specs=[
            rowblk(H), rowblk(H), rowblk(3 * H),
            full((H2, 3 * H)), full((H2, 3 * H)),
            full((H, H)), full((H, H)), full((1, H)),
        ],
        out_specs=[rowblk(H), rowblk(H)],
        out_shape=[node_t, node_t],
    )(mcL, mcR, iou0, U_iou[:H2], U_iou[H2:], W1a, W1b, b1_r)

    G = pl.kernel(
        _sc2_body,
        out_type=jax.ShapeDtypeStruct((E, H), F32),
        mesh=mesh,
        scratch_types=[
            pltpu.VMEM((1, K), jnp.int32),
            pltpu.VMEM((1, K), jnp.int32),
            pltpu.VMEM((1, K), jnp.int32),
            pltpu.VMEM((1, K), jnp.int32),
            pltpu.VMEM((K, H), F32),
            pltpu.VMEM((K, H), F32),
            pltpu.VMEM((K, H), F32),
            pltpu.VMEM((K, H), F32),
            pltpu.SemaphoreType.DMA,
            pltpu.SemaphoreType.DMA,
            pltpu.SemaphoreType.DMA,
            pltpu.SemaphoreType.DMA,
            pltpu.SemaphoreType.DMA,
            pltpu.SemaphoreType.DMA,
        ],
    )(P, Q, src, dst)

    RG = 4000
    out8 = pl.pallas_call(
        _p5_body,
        grid=(E // RG,),
        in_specs=[
            pl.BlockSpec((RG, H), lambda i: (i, 0)),
            full((H, OUTP)), full((1, OUTP)),
        ],
        out_specs=pl.BlockSpec((RG, OUTP), lambda i: (i, 0)),
        out_shape=jax.ShapeDtypeStruct((E, OUTP), F32),
    )(G, W2p, b2p)

    return out8[:, :3]


# speedup vs baseline: 5.1485x; 1.2806x over previous
import jax
import jax.numpy as jnp
from jax import lax
from jax.experimental import pallas as pl
from jax.experimental.pallas import tpu as pltpu
from jax.experimental.pallas import tpu_sc as plsc

F32 = jnp.float32
NC, NS, LANES = 2, 16, 16
K = 80


def _p1_body(x_ref, Wf_ref, Uf_ref, bf_ref, Wiou_ref, biou_ref,
             xbl_ref, xbr_ref, ap_ref, iou0_ref):
    xv = x_ref[...]
    H2 = xv.shape[1] // 2
    bv = jnp.dot(xv, Uf_ref[...], preferred_element_type=F32)
    xbl_ref[:, :H2] = xv[:, :H2]
    xbl_ref[:, H2:] = bv[:, :H2]
    xbr_ref[:, :H2] = xv[:, H2:]
    xbr_ref[:, H2:] = bv[:, H2:]
    ap_ref[...] = jnp.dot(xv, Wf_ref[...], preferred_element_type=F32) + bf_ref[...]
    iou0_ref[...] = jnp.dot(xv, Wiou_ref[...], preferred_element_type=F32) + biou_ref[...]


def _p3_body(mcl_ref, mcr_ref, iou0_ref, UiouL_ref, UiouR_ref,
             W1a_ref, W1b_ref, b1_ref, p_ref, q_ref):
    H = mcl_ref.shape[1]
    H2 = H // 2
    mcl = mcl_ref[...]
    mcr = mcr_ref[...]
    c_ref = jnp.concatenate([mcl[:, H2:], mcr[:, H2:]], axis=1)
    iou = (iou0_ref[...]
           + jnp.dot(mcl[:, :H2], UiouL_ref[...], preferred_element_type=F32)
           + jnp.dot(mcr[:, :H2], UiouR_ref[...], preferred_element_type=F32))
    i = jax.nn.sigmoid(iou[:, :H])
    o = jax.nn.sigmoid(iou[:, H:2 * H])
    u = jnp.tanh(iou[:, 2 * H:])
    c = i * u + c_ref
    h = o * jnp.tanh(c)
    p_ref[...] = jnp.dot(h, W1a_ref[...], preferred_element_type=F32)
    q_ref[...] = jnp.dot(h, W1b_ref[...], preferred_element_type=F32) + b1_ref[...]


def _p5_body(g_ref, W2_ref, b2_ref, out_ref):
    out_ref[...] = jnp.dot(jnp.tanh(g_ref[...]), W2_ref[...],
                           preferred_element_type=F32) + b2_ref[...]


def _sc1_body(xbl_hbm, xbr_hbm, ap_hbm, src_hbm, dst_hbm,
              mcl_hbm, mcr_hbm, *scr):
    idx_s = scr[0:4]
    idx_d = scr[4:8]
    xb_v = scr[8:10]
    a_v = scr[10:12]
    acc = scr[12]
    si = scr[13:17]
    sg = scr[17:19]
    ss = scr[19:21]
    cid = lax.axis_index("c")
    sid = lax.axis_index("s")
    E = src_hbm.shape[0]
    N, H = mcl_hbm.shape
    H2 = H // 2
    ept = E // NS
    nb = ept // K
    nch = N // K
    nq = H2 // LANES
    ebase = sid * ept
    acol = cid * H2

    def _zb(i, _):
        r = i // (2 * nq)
        q = i % (2 * nq)
        xb_v[0][r, pl.ds(q * LANES, LANES)] = jnp.zeros((LANES,), F32)
        return 0
    lax.fori_loop(0, K * 2 * nq, _zb, 0)

    def _zr(k, _):
        ch = k * NS + sid
        @pl.when(ch < nch)
        def _():
            pltpu.sync_copy(xb_v[0], acc.at[pl.ds(ch * K, K)])
        return 0
    lax.fori_loop(0, (nch + NS - 1) // NS, _zr, 0)
    plsc.subcore_barrier()

    def issue_idx(b, q):
        off = ebase + b * K
        pltpu.async_copy(src_hbm.at[pl.ds(off, K)], idx_s[q].at[0], si[q])
        pltpu.async_copy(dst_hbm.at[pl.ds(off, K)], idx_d[q].at[0], si[q])

    def wait_idx(q):
        pltpu.make_async_copy(src_hbm.at[pl.ds(0, K)], idx_s[q].at[0], si[q]).wait()
        pltpu.make_async_copy(dst_hbm.at[pl.ds(0, K)], idx_d[q].at[0], si[q]).wait()

    def issue_gathers(dp, q):
        pltpu.async_copy(ap_hbm.at[idx_d[q].at[0]], a_v[dp], sg[dp])
        @pl.when(cid == 0)
        def _():
            pltpu.async_copy(xbl_hbm.at[idx_s[q].at[0]], xb_v[dp], sg[dp])
        @pl.when(cid == 1)
        def _():
            pltpu.async_copy(xbr_hbm.at[idx_s[q].at[0]], xb_v[dp], sg[dp])

    def wait_gathers(dp, q):
        pltpu.make_async_copy(ap_hbm.at[idx_d[q].at[0]], a_v[dp], sg[dp]).wait()
        pltpu.make_async_copy(xbl_hbm.at[idx_s[q].at[0]], xb_v[dp], sg[dp]).wait()

    def issue_scatters(dp, q):
        pltpu.async_copy(xb_v[dp], acc.at[idx_d[q].at[0]], ss[dp], add=True)

    def wait_scatters(dp):
        pltpu.make_async_copy(xb_v[dp], acc.at[idx_d[0].at[0]], ss[dp]).wait()

    def compute(dp):
        @plsc.parallel_loop(0, K)
        def _ed(j):
            for q in range(nq):
                slx = pl.ds(q * LANES, LANES)
                slb = pl.ds(H2 + q * LANES, LANES)
                z = a_v[dp][j, pl.ds(acol + q * LANES, LANES)] + xb_v[dp][j, slb]
                f = 1.0 / (1.0 + jnp.exp(-z))
                xb_v[dp][j, slb] = f * xb_v[dp][j, slx]

    def phase(b, p):
        bt = jnp.int32(b)
        dp = p % 2
        wait_gathers(dp, p % 4)

        @pl.when(bt + 2 < nb)
        def _():
            issue_idx(bt + 2, (p + 2) % 4)

        @pl.when(bt + 1 < nb)
        def _():
            wait_idx((p + 1) % 4)
            @pl.when(bt >= 1)
            def _():
                wait_scatters(1 - dp)
            issue_gathers(1 - dp, (p + 1) % 4)

        compute(dp)
        issue_scatters(dp, p % 4)

    issue_idx(0, 0)
    wait_idx(0)
    issue_gathers(0, 0)
    issue_idx(1, 1)

    def _g(g, _):
        for p in (0, 1, 2, 3):
            phase(4 * g + p, p)
        return 0
    lax.fori_loop(0, (nb - 2) // 4, _g, 0)
    phase(nb - 2, 0)
    phase(nb - 1, 1)
    wait_scatters(0)
    wait_scatters(1)
    plsc.subcore_barrier()

    def _ex(k, _):
        ch = k * NS + sid
        @pl.when(ch < nch)
        def _():
            rows = pl.ds(ch * K, K)
            pltpu.sync_copy(acc.at[rows], xb_v[0])

            @pl.when(cid == 0)
            def _():
                pltpu.sync_copy(xb_v[0], mcl_hbm.at[rows])

            @pl.when(cid == 1)
            def _():
                pltpu.sync_copy(xb_v[0], mcr_hbm.at[rows])
        return 0
    lax.fori_loop(0, (nch + NS - 1) // NS, _ex, 0)


def _sc2_body(p_hbm, q_hbm, src_hbm, dst_hbm, g_hbm,
              idx_s0, idx_d0, idx_s1, idx_d1,
              p_v0, p_v1, q_v0, q_v1,
              sg0, sg1, sw0, sw1, si0, si1):
    cid = lax.axis_index("c")
    sid = lax.axis_index("s")
    wid = sid * NC + cid
    E = src_hbm.shape[0]
    H = p_hbm.shape[1]
    ept = E // (NC * NS)
    nb = ept // K
    nq = H // LANES
    ebase = wid * ept
    idx_s = (idx_s0, idx_s1)
    idx_d = (idx_d0, idx_d1)
    p_v = (p_v0, p_v1)
    q_v = (q_v0, q_v1)
    sg = (sg0, sg1)
    sw = (sw0, sw1)
    si = (si0, si1)

    def issue_idx(b, p):
        off = ebase + b * K
        pltpu.async_copy(src_hbm.at[pl.ds(off, K)], idx_s[p].at[0], si[p])
        pltpu.async_copy(dst_hbm.at[pl.ds(off, K)], idx_d[p].at[0], si[p])

    def wait_idx(p):
        pltpu.make_async_copy(src_hbm.at[pl.ds(0, K)], idx_s[p].at[0], si[p]).wait()
        pltpu.make_async_copy(dst_hbm.at[pl.ds(0, K)], idx_d[p].at[0], si[p]).wait()

    def issue_gathers(p):
        pltpu.async_copy(p_hbm.at[idx_s[p].at[0]], p_v[p], sg[p])
        pltpu.async_copy(q_hbm.at[idx_d[p].at[0]], q_v[p], sg[p])

    def wait_gathers(p):
        pltpu.make_async_copy(p_hbm.at[idx_s[p].at[0]], p_v[p], sg[p]).wait()
        pltpu.make_async_copy(q_hbm.at[idx_d[p].at[0]], q_v[p], sg[p]).wait()

    def issue_write(b, p):
        pltpu.async_copy(p_v[p], g_hbm.at[pl.ds(ebase + b * K, K)], sw[p])

    def wait_write(p):
        pltpu.make_async_copy(p_v[p], g_hbm.at[pl.ds(ebase, K)], sw[p]).wait()

    def compute(p):
        @plsc.parallel_loop(0, K)
        def _ed(j):
            for q in range(nq):
                sl = pl.ds(q * LANES, LANES)
                p_v[p][j, sl] = p_v[p][j, sl] + q_v[p][j, sl]

    def phase(b, p):
        bt = jnp.int32(b)
        wait_gathers(p)

        @pl.when(bt + 2 < nb)
        def _():
            issue_idx(bt + 2, p)

        @pl.when(bt + 1 < nb)
        def _():
            wait_idx(1 - p)
            @pl.when(bt >= 1)
            def _():
                wait_write(1 - p)
            issue_gathers(1 - p)

        compute(p)
        issue_write(bt, p)

    issue_idx(0, 0)
    wait_idx(0)
    issue_gathers(0)
    issue_idx(1, 1)

    def _g(g, _):
        for p in (0, 1):
            phase(2 * g + p, p)
        return 0
    lax.fori_loop(0, (nb - 1) // 2, _g, 0)
    phase(nb - 1, 0)
    wait_write(0)
    wait_write(1)


def kernel(x, edge_index, W_iou, U_iou, b_iou, W_f, U_f, b_f,
           clf_W1, clf_b1, clf_W2, clf_b2):
    N, D = x.shape
    H = U_f.shape[0]
    E = edge_index.shape[1]
    src = edge_index[0]
    dst = edge_index[1]

    bf_r = b_f.reshape(1, H)
    biou_r = b_iou.reshape(1, 3 * H)
    b1_r = clf_b1.reshape(1, -1)
    W1a = clf_W1[:H]
    W1b = clf_W1[H:]
    OUTP = 8
    W2p = jnp.pad(clf_W2, ((0, 0), (0, OUTP - clf_W2.shape[1])))
    b2p = jnp.pad(clf_b2, (0, OUTP - clf_b2.shape[0])).reshape(1, OUTP)

    R1 = 1000
    H2 = H // 2
    full = lambda s: pl.BlockSpec(s, lambda i: (0, 0))
    rowblk = lambda w: pl.BlockSpec((R1, w), lambda i: (i, 0))
    node_t = jax.ShapeDtypeStruct((N, H), F32)
    XBL, XBR, Ap, iou0 = pl.pallas_call(
        _p1_body,
        grid=(N // R1,),
        in_specs=[
            rowblk(D),
            full((D, H)), full((H, H)), full((1, H)),
            full((D, 3 * H)), full((1, 3 * H)),
        ],
        out_specs=[rowblk(H)] * 3 + [rowblk(3 * H)],
        out_shape=[node_t] * 3 + [jax.ShapeDtypeStruct((N, 3 * H), F32)],
    )(x, W_f, U_f, bf_r, W_iou, biou_r)

    mesh = plsc.VectorSubcoreMesh(core_axis_name="c", subcore_axis_name="s",
                                  num_cores=NC, num_subcores=NS)
    mcL, mcR = pl.kernel(
        _sc1_body,
        out_type=(node_t, node_t),
        mesh=mesh,
        scratch_types=(
            [pltpu.VMEM((1, K), jnp.int32)] * 8
            + [pltpu.VMEM((K, H), F32)] * 4
            + [pltpu.VMEM_SHARED((N, H), F32)]
            + [pltpu.SemaphoreType.DMA] * 8
        ),
    )(XBL, XBR, Ap, src, dst)

    P, Q = pl.pallas_call(
        _p3_body,
        grid=(N // R1,),
        in_specs=[
            rowblk(H), rowblk(H), rowblk(3 * H),
            full((H2, 3 * H)), full((H2, 3 * H)),
            full((H, H)), full((H, H)), full((1, H)),
        ],
        out_specs=[rowblk(H), rowblk(H)],
        out_shape=[node_t, node_t],
    )(mcL, mcR, iou0, U_iou[:H2], U_iou[H2:], W1a, W1b, b1_r)

    G = pl.kernel(
        _sc2_body,
        out_type=jax.ShapeDtypeStruct((E, H), F32),
        mesh=mesh,
        scratch_types=[
            pltpu.VMEM((1, K), jnp.int32),
            pltpu.VMEM((1, K), jnp.int32),
            pltpu.VMEM((1, K), jnp.int32),
            pltpu.VMEM((1, K), jnp.int32),
            pltpu.VMEM((K, H), F32),
            pltpu.VMEM((K, H), F32),
            pltpu.VMEM((K, H), F32),
            pltpu.VMEM((K, H), F32),
            pltpu.SemaphoreType.DMA,
            pltpu.SemaphoreType.DMA,
            pltpu.SemaphoreType.DMA,
            pltpu.SemaphoreType.DMA,
            pltpu.SemaphoreType.DMA,
            pltpu.SemaphoreType.DMA,
        ],
    )(P, Q, src, dst)

    RG = 4000
    out8 = pl.pallas_call(
        _p5_body,
        grid=(E // RG,),
        in_specs=[
            pl.BlockSpec((RG, H), lambda i: (i, 0)),
            full((H, OUTP)), full((1, OUTP)),
        ],
        out_specs=pl.BlockSpec((RG, OUTP), lambda i: (i, 0)),
        out_shape=jax.ShapeDtypeStruct((E, OUTP), F32),
    )(G, W2p, b2p)

    return out8[:, :3]
